# Initial kernel scaffold; baseline (speedup 1.0000x reference)
import jax
import jax.numpy as jnp
from jax.experimental import pallas as pl


def kernel(hidden_states, gate_w, bias, w1, w3, w2, sh_wg, sh_wu, sh_wd):
    raise NotImplementedError("write your pallas kernel here")


# speedup vs baseline: 1.7813x; 1.7813x over previous
import functools

import jax
import jax.numpy as jnp
from jax.experimental import pallas as pl
from jax.experimental.pallas import tpu as pltpu

E = 16
TOPK = 2
H = 2048
FF = 1024
SF = 1024
T = 2048
A = T * TOPK
TM = 256
NT = (A + E * TM) // TM
NPAD = NT * TM

TMS = 256


def _silu(g):
    return g * (1.0 / (1.0 + jnp.exp(-g)))


def _moe_tile_kernel(eot_ref, cf_ref, xs_ref, w1_ref, w3_ref, w2_ref, ys_ref):
    i = pl.program_id(0)

    @pl.when(cf_ref[i] > 0)
    def _():
        xb = xs_ref[...].astype(jnp.bfloat16)
        w1b = w1_ref[0].astype(jnp.bfloat16)
        w3b = w3_ref[0].astype(jnp.bfloat16)
        g = jax.lax.dot_general(xb, w1b, (((1,), (1,)), ((), ())),
                                preferred_element_type=jnp.float32)
        u = jax.lax.dot_general(xb, w3b, (((1,), (1,)), ((), ())),
                                preferred_element_type=jnp.float32)
        h = (_silu(g) * u).astype(jnp.bfloat16)
        w2b = w2_ref[0].astype(jnp.bfloat16)
        ys_ref[...] = jax.lax.dot_general(h, w2b, (((1,), (1,)), ((), ())),
                                          preferred_element_type=jnp.float32)


def _shared_kernel(x_ref, wg_ref, wu_ref, wd_ref, o_ref):
    xb = x_ref[...].astype(jnp.bfloat16)
    wg = wg_ref[...].astype(jnp.bfloat16)
    wu = wu_ref[...].astype(jnp.bfloat16)
    g = jax.lax.dot_general(xb, wg, (((1,), (1,)), ((), ())),
                            preferred_element_type=jnp.float32)
    u = jax.lax.dot_general(xb, wu, (((1,), (1,)), ((), ())),
                            preferred_element_type=jnp.float32)
    h = (_silu(g) * u).astype(jnp.bfloat16)
    wd = wd_ref[...].astype(jnp.bfloat16)
    o_ref[...] = jax.lax.dot_general(h, wd, (((1,), (1,)), ((), ())),
                                     preferred_element_type=jnp.float32)


def kernel(hidden_states, gate_w, bias, w1, w3, w2, sh_wg, sh_wu, sh_wd):
    x = hidden_states

    logits = x @ gate_w.T
    scores = jax.nn.sigmoid(logits)
    biased = scores + bias
    _, topk_idx = jax.lax.top_k(biased, TOPK)
    topk_w = jnp.take_along_axis(scores, topk_idx, axis=1)
    topk_w = topk_w / jnp.sum(topk_w, axis=1, keepdims=True)

    eid = topk_idx.reshape(-1).astype(jnp.int32)
    perm = jnp.argsort(eid, stable=True).astype(jnp.int32)
    eid_s = eid[perm]
    tok_s = perm // TOPK
    counts = jnp.bincount(eid, length=E).astype(jnp.int32)
    padded = ((counts + TM - 1) // TM) * TM
    offs = jnp.concatenate([jnp.zeros(1, jnp.int32),
                            jnp.cumsum(padded)[:-1].astype(jnp.int32)])
    csum = jnp.concatenate([jnp.zeros(1, jnp.int32),
                            jnp.cumsum(counts)[:-1].astype(jnp.int32)])
    rank = jnp.arange(A, dtype=jnp.int32) - csum[eid_s]
    dest = offs[eid_s] + rank
    ids_p = jnp.zeros(NPAD, jnp.int32).at[dest].set(tok_s)
    pos = jnp.zeros(A, jnp.int32).at[perm].set(dest).reshape(T, TOPK)

    tend = jnp.cumsum(padded // TM).astype(jnp.int32)
    tidx = jnp.arange(NT, dtype=jnp.int32)
    tile_expert = jnp.minimum(
        jnp.searchsorted(tend, tidx, side="right").astype(jnp.int32), E - 1)
    tile_valid = (tidx < tend[-1]).astype(jnp.int32)

    xs = x[ids_p]

    ys = pl.pallas_call(
        _moe_tile_kernel,
        grid_spec=pltpu.PrefetchScalarGridSpec(
            num_scalar_prefetch=2,
            grid=(NT,),
            in_specs=[
                pl.BlockSpec((TM, H), lambda i, eot, cf: (i, 0)),
                pl.BlockSpec((1, FF, H), lambda i, eot, cf: (eot[i], 0, 0)),
                pl.BlockSpec((1, FF, H), lambda i, eot, cf: (eot[i], 0, 0)),
                pl.BlockSpec((1, H, FF), lambda i, eot, cf: (eot[i], 0, 0)),
            ],
            out_specs=pl.BlockSpec((TM, H), lambda i, eot, cf: (i, 0)),
        ),
        out_shape=jax.ShapeDtypeStruct((NPAD, H), jnp.float32),
    )(tile_expert, tile_valid, xs, w1, w3, w2)

    shared = pl.pallas_call(
        _shared_kernel,
        grid=(T // TMS,),
        in_specs=[
            pl.BlockSpec((TMS, H), lambda i: (i, 0)),
            pl.BlockSpec((SF, H), lambda i: (0, 0)),
            pl.BlockSpec((SF, H), lambda i: (0, 0)),
            pl.BlockSpec((H, SF), lambda i: (0, 0)),
        ],
        out_specs=pl.BlockSpec((TMS, H), lambda i: (i, 0)),
        out_shape=jax.ShapeDtypeStruct((T, H), jnp.float32),
    )(x, sh_wg, sh_wu, sh_wd)

    gath = ys[pos.reshape(-1)].reshape(T, TOPK, H)
    return shared + jnp.sum(gath * topk_w[..., None], axis=1)


# speedup vs baseline: 1.9087x; 1.0715x over previous
import functools

import jax
import jax.numpy as jnp
from jax.experimental import pallas as pl
from jax.experimental.pallas import tpu as pltpu

E = 16
TOPK = 2
H = 2048
FF = 1024
SF = 1024
T = 2048
A = T * TOPK
TM = 256
NT = (A + E * TM) // TM
NPAD = NT * TM

TMS = 256


def _silu(g):
    return g * (1.0 / (1.0 + jnp.exp(-g)))


def _moe_tile_kernel(eot_ref, cf_ref, xs_ref, w1_ref, w3_ref, w2_ref, ys_ref):
    i = pl.program_id(0)

    @pl.when(cf_ref[i] > 0)
    def _():
        xb = xs_ref[...].astype(jnp.bfloat16)
        w1b = w1_ref[0].astype(jnp.bfloat16)
        w3b = w3_ref[0].astype(jnp.bfloat16)
        g = jax.lax.dot_general(xb, w1b, (((1,), (1,)), ((), ())),
                                preferred_element_type=jnp.float32)
        u = jax.lax.dot_general(xb, w3b, (((1,), (1,)), ((), ())),
                                preferred_element_type=jnp.float32)
        h = (_silu(g) * u).astype(jnp.bfloat16)
        w2b = w2_ref[0].astype(jnp.bfloat16)
        ys_ref[...] = jax.lax.dot_general(h, w2b, (((1,), (1,)), ((), ())),
                                          preferred_element_type=jnp.float32)


def _shared_kernel(x_ref, wg_ref, wu_ref, wd_ref, o_ref):
    xb = x_ref[...].astype(jnp.bfloat16)
    wg = wg_ref[...].astype(jnp.bfloat16)
    wu = wu_ref[...].astype(jnp.bfloat16)
    g = jax.lax.dot_general(xb, wg, (((1,), (1,)), ((), ())),
                            preferred_element_type=jnp.float32)
    u = jax.lax.dot_general(xb, wu, (((1,), (1,)), ((), ())),
                            preferred_element_type=jnp.float32)
    h = (_silu(g) * u).astype(jnp.bfloat16)
    wd = wd_ref[...].astype(jnp.bfloat16)
    o_ref[...] = jax.lax.dot_general(h, wd, (((1,), (1,)), ((), ())),
                                     preferred_element_type=jnp.float32)


def kernel(hidden_states, gate_w, bias, w1, w3, w2, sh_wg, sh_wu, sh_wd):
    x = hidden_states

    logits = x @ gate_w.T
    scores = jax.nn.sigmoid(logits)
    biased = scores + bias
    _, topk_idx = jax.lax.top_k(biased, TOPK)
    topk_w = jnp.take_along_axis(scores, topk_idx, axis=1)
    topk_w = topk_w / jnp.sum(topk_w, axis=1, keepdims=True)

    eid = topk_idx.reshape(-1).astype(jnp.int32)
    tok = jnp.arange(A, dtype=jnp.int32) // TOPK
    oh = (eid[:, None] == jnp.arange(E, dtype=jnp.int32)[None, :])
    cum = jnp.cumsum(oh.astype(jnp.int32), axis=0)
    counts = cum[-1]
    rank = jnp.take_along_axis(cum, eid[:, None], axis=1)[:, 0] - 1
    padded = ((counts + TM - 1) // TM) * TM
    offs = jnp.concatenate([jnp.zeros(1, jnp.int32),
                            jnp.cumsum(padded)[:-1].astype(jnp.int32)])
    dest = offs[eid] + rank
    pos = dest.reshape(T, TOPK)
    ids_p = jnp.zeros(NPAD, jnp.int32).at[dest].set(tok)

    tend = jnp.cumsum(padded // TM).astype(jnp.int32)
    tidx = jnp.arange(NT, dtype=jnp.int32)
    tile_expert = jnp.minimum(
        jnp.sum((tend[None, :] <= tidx[:, None]).astype(jnp.int32), axis=1),
        E - 1)
    tile_valid = (tidx < tend[-1]).astype(jnp.int32)

    xs = x[ids_p]

    ys = pl.pallas_call(
        _moe_tile_kernel,
        grid_spec=pltpu.PrefetchScalarGridSpec(
            num_scalar_prefetch=2,
            grid=(NT,),
            in_specs=[
                pl.BlockSpec((TM, H), lambda i, eot, cf: (i, 0)),
                pl.BlockSpec((1, FF, H), lambda i, eot, cf: (eot[i], 0, 0)),
                pl.BlockSpec((1, FF, H), lambda i, eot, cf: (eot[i], 0, 0)),
                pl.BlockSpec((1, H, FF), lambda i, eot, cf: (eot[i], 0, 0)),
            ],
            out_specs=pl.BlockSpec((TM, H), lambda i, eot, cf: (i, 0)),
        ),
        out_shape=jax.ShapeDtypeStruct((NPAD, H), jnp.float32),
    )(tile_expert, tile_valid, xs, w1, w3, w2)

    shared = pl.pallas_call(
        _shared_kernel,
        grid=(T // TMS,),
        in_specs=[
            pl.BlockSpec((TMS, H), lambda i: (i, 0)),
            pl.BlockSpec((SF, H), lambda i: (0, 0)),
            pl.BlockSpec((SF, H), lambda i: (0, 0)),
            pl.BlockSpec((H, SF), lambda i: (0, 0)),
        ],
        out_specs=pl.BlockSpec((TMS, H), lambda i: (i, 0)),
        out_shape=jax.ShapeDtypeStruct((T, H), jnp.float32),
    )(x, sh_wg, sh_wu, sh_wd)

    gath = ys[pos.reshape(-1)].reshape(T, TOPK, H)
    return shared + jnp.sum(gath * topk_w[..., None], axis=1)
